# Initial kernel scaffold; baseline (speedup 1.0000x reference)
import functools

import jax
import jax.numpy as jnp
from jax import lax
from jax.experimental import pallas as pl
from jax.experimental.pallas import tpu as pltpu
from jax.experimental.pallas import tpu_sc as plsc

B = 16384
HIST = 50
D = 32

_info = plsc.get_sparse_core_info()
NC, NS, L = _info.num_cores, _info.num_subcores, _info.num_lanes
NW = NC * NS
RPW = B // NW
C = 64
NCHUNK = RPW // C
HPC = C * HIST
GSZ = 128
NGS = HPC // GSZ

_mesh = plsc.VectorSubcoreMesh(core_axis_name="c", subcore_axis_name="s")


@functools.partial(
    pl.kernel,
    out_type=jax.ShapeDtypeStruct((B, 4 * D), jnp.float32),
    mesh=_mesh,
    scratch_types=[
        pltpu.VMEM((NGS, GSZ), jnp.int32),
        pltpu.VMEM((C,), jnp.int32),
        pltpu.VMEM((C,), jnp.int32),
        pltpu.VMEM((C,), jnp.int32),
        pltpu.VMEM((HPC, D), jnp.float32),
        pltpu.VMEM((C, D), jnp.float32),
        pltpu.VMEM((C, D), jnp.float32),
        pltpu.VMEM((C, D), jnp.float32),
        pltpu.VMEM((C, D), jnp.float32),
        pltpu.SemaphoreType.DMA,
    ],
)
def _sc_kernel(hist2d, uid, iid, cid, w_hist, w_user, w_item, w_cat, out,
               hidx, uidx, iidx, cidx, hist_v, urow, irow, crow, mean_v,
               sem):
    wid = lax.axis_index("s") * NC + lax.axis_index("c")
    base = wid * RPW
    inv = jnp.float32(1.0 / HIST)

    def chunk(k, carry):
        cb = base + k * C
        pltpu.sync_copy(hist2d.at[pl.ds(cb * HIST // GSZ, NGS)], hidx)
        pltpu.sync_copy(uid.at[pl.ds(cb, C)], uidx)
        pltpu.sync_copy(iid.at[pl.ds(cb, C)], iidx)
        pltpu.sync_copy(cid.at[pl.ds(cb, C)], cidx)

        cps = []
        for s in range(NGS):
            cps.append(pltpu.async_copy(
                w_hist.at[hidx.at[s]],
                hist_v.at[pl.ds(s * GSZ, GSZ)], sem))
        cps.append(pltpu.async_copy(w_user.at[uidx], urow, sem))
        cps.append(pltpu.async_copy(w_item.at[iidx], irow, sem))
        cps.append(pltpu.async_copy(w_cat.at[cidx], crow, sem))
        for cp in cps:
            cp.wait()

        def row(r, carry2):
            h0 = r * HIST
            a0 = jnp.zeros((L,), jnp.float32)
            a1 = jnp.zeros((L,), jnp.float32)
            b0 = jnp.zeros((L,), jnp.float32)
            b1 = jnp.zeros((L,), jnp.float32)
            for j in range(0, HIST, 2):
                a0 = a0 + hist_v[h0 + j, pl.ds(0, L)]
                a1 = a1 + hist_v[h0 + j, pl.ds(L, L)]
                b0 = b0 + hist_v[h0 + j + 1, pl.ds(0, L)]
                b1 = b1 + hist_v[h0 + j + 1, pl.ds(L, L)]
            mean_v[r, pl.ds(0, L)] = (a0 + b0) * inv
            mean_v[r, pl.ds(L, L)] = (a1 + b1) * inv
            return carry2

        lax.fori_loop(0, C, row, 0)

        pltpu.sync_copy(mean_v, out.at[pl.ds(cb, C), pl.ds(0, D)])
        pltpu.sync_copy(urow, out.at[pl.ds(cb, C), pl.ds(D, D)])
        pltpu.sync_copy(irow, out.at[pl.ds(cb, C), pl.ds(2 * D, D)])
        pltpu.sync_copy(crow, out.at[pl.ds(cb, C), pl.ds(3 * D, D)])
        return carry

    lax.fori_loop(0, NCHUNK, chunk, 0)


def kernel(user_id, item_id, category_id, hist_item_id, W_hist, W_user,
           W_item, W_cat):
    hist2d = hist_item_id.astype(jnp.int32).reshape(B * HIST // GSZ, GSZ)
    uid = user_id.astype(jnp.int32).reshape(B)
    iid = item_id.astype(jnp.int32).reshape(B)
    cid = category_id.astype(jnp.int32).reshape(B)
    return _sc_kernel(hist2d, uid, iid, cid, W_hist, W_user, W_item, W_cat)


# speedup vs baseline: 1.2593x; 1.2593x over previous
import functools

import jax
import jax.numpy as jnp
from jax import lax
from jax.experimental import pallas as pl
from jax.experimental.pallas import tpu as pltpu
from jax.experimental.pallas import tpu_sc as plsc

B = 16384
HIST = 50
D = 32

_info = plsc.get_sparse_core_info()
NC, NS, L = _info.num_cores, _info.num_subcores, _info.num_lanes
NW = NC * NS
RPW = B // NW
C = 64
NCHUNK = RPW // C
HPC = C * HIST
GSZ = 128
NGS = HPC // GSZ

_mesh = plsc.VectorSubcoreMesh(core_axis_name="c", subcore_axis_name="s")


@functools.partial(
    pl.kernel,
    out_type=jax.ShapeDtypeStruct((B, 4 * D), jnp.float32),
    mesh=_mesh,
    compiler_params=pltpu.CompilerParams(use_tc_tiling_on_sc=False),
    scratch_types=[
        pltpu.VMEM((HPC,), jnp.int32),
        pltpu.VMEM((C,), jnp.int32),
        pltpu.VMEM((C,), jnp.int32),
        pltpu.VMEM((C,), jnp.int32),
        pltpu.VMEM((HPC, D), jnp.float32),
        pltpu.VMEM((C, D), jnp.float32),
        pltpu.VMEM((C, D), jnp.float32),
        pltpu.VMEM((C, D), jnp.float32),
        pltpu.VMEM((C, 4 * D), jnp.float32),
        pltpu.SemaphoreType.DMA,
    ],
)
def _sc_kernel(hids, uid, iid, cid, w_hist, w_user, w_item, w_cat, out,
               hidx, uidx, iidx, cidx, hist_v, urow, irow, crow, outbuf,
               sem):
    wid = lax.axis_index("s") * NC + lax.axis_index("c")
    base = wid * RPW
    inv = jnp.float32(1.0 / HIST)

    def chunk(k, carry):
        cb = base + k * C
        pltpu.sync_copy(hids.at[pl.ds(cb * HIST, HPC)], hidx)
        pltpu.sync_copy(uid.at[pl.ds(cb, C)], uidx)
        pltpu.sync_copy(iid.at[pl.ds(cb, C)], iidx)
        pltpu.sync_copy(cid.at[pl.ds(cb, C)], cidx)

        cps = []
        for s in range(NGS):
            cps.append(pltpu.async_copy(
                w_hist.at[hidx.at[pl.ds(s * GSZ, GSZ)]],
                hist_v.at[pl.ds(s * GSZ, GSZ)], sem))
        cps.append(pltpu.async_copy(w_user.at[uidx], urow, sem))
        cps.append(pltpu.async_copy(w_item.at[iidx], irow, sem))
        cps.append(pltpu.async_copy(w_cat.at[cidx], crow, sem))
        for cp in cps:
            cp.wait()

        def row(r, carry2):
            h0 = r * HIST
            a0 = jnp.zeros((L,), jnp.float32)
            a1 = jnp.zeros((L,), jnp.float32)
            b0 = jnp.zeros((L,), jnp.float32)
            b1 = jnp.zeros((L,), jnp.float32)
            for j in range(0, HIST, 2):
                a0 = a0 + hist_v[h0 + j, pl.ds(0, L)]
                a1 = a1 + hist_v[h0 + j, pl.ds(L, L)]
                b0 = b0 + hist_v[h0 + j + 1, pl.ds(0, L)]
                b1 = b1 + hist_v[h0 + j + 1, pl.ds(L, L)]
            outbuf[r, pl.ds(0, L)] = (a0 + b0) * inv
            outbuf[r, pl.ds(L, L)] = (a1 + b1) * inv
            outbuf[r, pl.ds(2 * L, L)] = urow[r, pl.ds(0, L)]
            outbuf[r, pl.ds(3 * L, L)] = urow[r, pl.ds(L, L)]
            outbuf[r, pl.ds(4 * L, L)] = irow[r, pl.ds(0, L)]
            outbuf[r, pl.ds(5 * L, L)] = irow[r, pl.ds(L, L)]
            outbuf[r, pl.ds(6 * L, L)] = crow[r, pl.ds(0, L)]
            outbuf[r, pl.ds(7 * L, L)] = crow[r, pl.ds(L, L)]
            return carry2

        lax.fori_loop(0, C, row, 0)

        pltpu.sync_copy(outbuf, out.at[pl.ds(cb, C)])
        return carry

    lax.fori_loop(0, NCHUNK, chunk, 0)


def kernel(user_id, item_id, category_id, hist_item_id, W_hist, W_user,
           W_item, W_cat):
    hids = hist_item_id.astype(jnp.int32).reshape(B * HIST)
    uid = user_id.astype(jnp.int32).reshape(B)
    iid = item_id.astype(jnp.int32).reshape(B)
    cid = category_id.astype(jnp.int32).reshape(B)
    return _sc_kernel(hids, uid, iid, cid, W_hist, W_user, W_item, W_cat)
